# baseline (device time: 24781 ns/iter reference)
import jax
import jax.numpy as jnp
from jax import lax
from jax.experimental import pallas as pl
from jax.experimental.pallas import tpu as pltpu


def kernel(O, Wo):
    B, S, H, D = O.shape
    K = H * D
    N = Wo.shape[1]
    S_half = S // 2

    def body(
        o_ref,
        wo_ref,
        out_ref,
        o_flat,
        sendx_buf,
        recvx_buf,
        recvy_buf,
        recvz_buf,
        sendx_sems,
        recvx_sems,
        fwd_send_sems,
        fwd_recv_sems,
    ):
        my_x = lax.axis_index("x")
        my_y = lax.axis_index("y")
        my_z = lax.axis_index("z")
        x_nbr = (1 - my_x, my_y, my_z)
        y_nbr = (my_x, 1 - my_y, my_z)
        z_nbr = (my_x, my_y, 1 - my_z)

        b1 = 2 * my_y + my_z
        b2 = 3 - b1
        b_y = 2 * (1 - my_y) + my_z
        b_z = 2 * my_y + (1 - my_z)

        barrier_sem = pltpu.get_barrier_semaphore()
        for nbr in (x_nbr, y_nbr, z_nbr):
            pl.semaphore_signal(
                barrier_sem, inc=1, device_id=nbr,
                device_id_type=pl.DeviceIdType.MESH,
            )
        pl.semaphore_wait(barrier_sem, 3)

        other_start = (1 - my_x) * S_half
        my_start = my_x * S_half

        wo = wo_ref[:, :]

        Sh2 = S_half // 2

        def flatten(slot, b, start):
            for h in range(H):
                o_flat[slot, :, h * D:(h + 1) * D] = o_ref[
                    pl.ds(b, 1), pl.ds(start, S_half), h, :
                ].reshape(S_half, D)

        def x_rdma(slot):
            if slot == 0:
                src = sendx_buf.at[0, pl.ds(0, Sh2)]
                dst = recvx_buf.at[0, pl.ds(0, Sh2)]
            elif slot == 1:
                src = sendx_buf.at[0, pl.ds(Sh2, Sh2)]
                dst = recvx_buf.at[0, pl.ds(Sh2, Sh2)]
            else:
                src = sendx_buf.at[1]
                dst = recvx_buf.at[1]
            return pltpu.make_async_remote_copy(
                src_ref=src,
                dst_ref=dst,
                send_sem=sendx_sems.at[slot],
                recv_sem=recvx_sems.at[slot],
                device_id=x_nbr,
                device_id_type=pl.DeviceIdType.MESH,
            )

        def fwd_rdma(direction, half):
            nbr = y_nbr if direction == 0 else z_nbr
            dst_buf = recvy_buf if direction == 0 else recvz_buf
            slot = 2 * direction + half
            rows = pl.ds(half * Sh2, Sh2)
            return pltpu.make_async_remote_copy(
                src_ref=recvx_buf.at[0, rows],
                dst_ref=dst_buf.at[rows],
                send_sem=fwd_send_sems.at[slot],
                recv_sem=fwd_recv_sems.at[slot],
                device_id=nbr,
                device_id_type=pl.DeviceIdType.MESH,
            )

        flatten(0, b1, other_start)
        sendx_buf[0, :, :] = jnp.dot(
            o_flat[0], wo, preferred_element_type=jnp.float32
        ).astype(jnp.bfloat16)
        x_rdma(0).start()
        x_rdma(1).start()

        flatten(1, b2, other_start)
        sendx_buf[1, :, :] = jnp.dot(
            o_flat[1], wo, preferred_element_type=jnp.float32
        ).astype(jnp.bfloat16)
        x_rdma(2).start()

        for b in range(B):
            flatten(b % 2, b, my_start)
            out_ref[b, :, :] = jnp.dot(
                o_flat[b % 2], wo, preferred_element_type=jnp.float32
            )

        x_rdma(0).wait_recv()
        fwd_rdma(0, 0).start()
        fwd_rdma(1, 0).start()
        x_rdma(1).wait_recv()
        fwd_rdma(0, 1).start()
        fwd_rdma(1, 1).start()

        def accum(b, contrib_bf16):
            cur = out_ref[pl.ds(b, 1), :, :]
            out_ref[pl.ds(b, 1), :, :] = cur + contrib_bf16.astype(
                jnp.float32
            ).reshape(1, S_half, N)

        accum(b1, recvx_buf[0, :, :])

        fwd_rdma(0, 0).wait_recv()
        fwd_rdma(0, 1).wait_recv()
        accum(b_y, recvy_buf[:, :])

        fwd_rdma(1, 0).wait_recv()
        fwd_rdma(1, 1).wait_recv()
        accum(b_z, recvz_buf[:, :])

        x_rdma(2).wait_recv()
        accum(b2, recvx_buf[1, :, :])

        for slot in range(3):
            x_rdma(slot).wait_send()
        for direction in range(2):
            for half in range(2):
                fwd_rdma(direction, half).wait_send()

    return pl.pallas_call(
        body,
        out_shape=jax.ShapeDtypeStruct((B, S_half, N), jnp.float32),
        in_specs=[
            pl.BlockSpec(memory_space=pltpu.VMEM),
            pl.BlockSpec(memory_space=pltpu.VMEM),
        ],
        out_specs=pl.BlockSpec(memory_space=pltpu.VMEM),
        scratch_shapes=[
            pltpu.VMEM((2, S_half, K), jnp.float32),
            pltpu.VMEM((2, S_half, N), jnp.bfloat16),
            pltpu.VMEM((2, S_half, N), jnp.bfloat16),
            pltpu.VMEM((S_half, N), jnp.bfloat16),
            pltpu.VMEM((S_half, N), jnp.bfloat16),
            pltpu.SemaphoreType.DMA((3,)),
            pltpu.SemaphoreType.DMA((3,)),
            pltpu.SemaphoreType.DMA((4,)),
            pltpu.SemaphoreType.DMA((4,)),
        ],
        compiler_params=pltpu.CompilerParams(collective_id=0),
    )(O, Wo)


# device time: 22926 ns/iter; 1.0809x vs baseline; 1.0809x over previous
import jax
import jax.numpy as jnp
from jax import lax
from jax.experimental import pallas as pl
from jax.experimental.pallas import tpu as pltpu


def kernel(O, Wo):
    B, S, H, D = O.shape
    K = H * D
    N = Wo.shape[1]
    S_half = S // 2

    O2 = O.astype(jnp.bfloat16).reshape(B, S, K)

    def body(
        o_ref,
        wo_ref,
        out_ref,
        sendx_buf,
        recvx_buf,
        recvy_buf,
        recvz_buf,
        sendx_sems,
        recvx_sems,
        fwd_send_sems,
        fwd_recv_sems,
    ):
        my_x = lax.axis_index("x")
        my_y = lax.axis_index("y")
        my_z = lax.axis_index("z")
        x_nbr = (1 - my_x, my_y, my_z)
        y_nbr = (my_x, 1 - my_y, my_z)
        z_nbr = (my_x, my_y, 1 - my_z)

        b1 = 2 * my_y + my_z
        b2 = 3 - b1
        b_y = 2 * (1 - my_y) + my_z
        b_z = 2 * my_y + (1 - my_z)

        barrier_sem = pltpu.get_barrier_semaphore()
        for nbr in (x_nbr, y_nbr, z_nbr):
            pl.semaphore_signal(
                barrier_sem, inc=1, device_id=nbr,
                device_id_type=pl.DeviceIdType.MESH,
            )
        pl.semaphore_wait(barrier_sem, 3)

        other_start = (1 - my_x) * S_half
        my_start = my_x * S_half

        wo = wo_ref[:, :].astype(jnp.bfloat16)

        Sh2 = S_half // 2

        def x_rdma(slot):
            if slot == 0:
                src = sendx_buf.at[0, pl.ds(0, Sh2)]
                dst = recvx_buf.at[0, pl.ds(0, Sh2)]
            elif slot == 1:
                src = sendx_buf.at[0, pl.ds(Sh2, Sh2)]
                dst = recvx_buf.at[0, pl.ds(Sh2, Sh2)]
            else:
                src = sendx_buf.at[1]
                dst = recvx_buf.at[1]
            return pltpu.make_async_remote_copy(
                src_ref=src,
                dst_ref=dst,
                send_sem=sendx_sems.at[slot],
                recv_sem=recvx_sems.at[slot],
                device_id=x_nbr,
                device_id_type=pl.DeviceIdType.MESH,
            )

        def fwd_rdma(direction, half):
            nbr = y_nbr if direction == 0 else z_nbr
            dst_buf = recvy_buf if direction == 0 else recvz_buf
            slot = 2 * direction + half
            rows = pl.ds(half * Sh2, Sh2)
            return pltpu.make_async_remote_copy(
                src_ref=recvx_buf.at[0, rows],
                dst_ref=dst_buf.at[rows],
                send_sem=fwd_send_sems.at[slot],
                recv_sem=fwd_recv_sems.at[slot],
                device_id=nbr,
                device_id_type=pl.DeviceIdType.MESH,
            )

        o_q1 = o_ref[pl.ds(b1, 1), pl.ds(other_start, S_half), :].reshape(
            S_half, K
        )
        sendx_buf[0, :, :] = jnp.dot(
            o_q1, wo, preferred_element_type=jnp.float32
        ).astype(jnp.bfloat16)
        x_rdma(0).start()
        x_rdma(1).start()

        o_q2 = o_ref[pl.ds(b2, 1), pl.ds(other_start, S_half), :].reshape(
            S_half, K
        )
        sendx_buf[1, :, :] = jnp.dot(
            o_q2, wo, preferred_element_type=jnp.float32
        ).astype(jnp.bfloat16)
        x_rdma(2).start()

        for b in range(B):
            o_mine = o_ref[b, pl.ds(my_start, S_half), :]
            out_ref[b, :, :] = jnp.dot(
                o_mine, wo, preferred_element_type=jnp.float32
            )

        x_rdma(0).wait_recv()
        fwd_rdma(0, 0).start()
        fwd_rdma(1, 0).start()
        x_rdma(1).wait_recv()
        fwd_rdma(0, 1).start()
        fwd_rdma(1, 1).start()

        def accum(b, contrib_bf16):
            cur = out_ref[pl.ds(b, 1), :, :]
            out_ref[pl.ds(b, 1), :, :] = cur + contrib_bf16.astype(
                jnp.float32
            ).reshape(1, S_half, N)

        accum(b1, recvx_buf[0, :, :])

        fwd_rdma(0, 0).wait_recv()
        fwd_rdma(0, 1).wait_recv()
        accum(b_y, recvy_buf[:, :])

        fwd_rdma(1, 0).wait_recv()
        fwd_rdma(1, 1).wait_recv()
        accum(b_z, recvz_buf[:, :])

        x_rdma(2).wait_recv()
        accum(b2, recvx_buf[1, :, :])

        for slot in range(3):
            x_rdma(slot).wait_send()
        for direction in range(2):
            for half in range(2):
                fwd_rdma(direction, half).wait_send()

    return pl.pallas_call(
        body,
        out_shape=jax.ShapeDtypeStruct((B, S_half, N), jnp.float32),
        in_specs=[
            pl.BlockSpec(memory_space=pltpu.VMEM),
            pl.BlockSpec(memory_space=pltpu.VMEM),
        ],
        out_specs=pl.BlockSpec(memory_space=pltpu.VMEM),
        scratch_shapes=[
            pltpu.VMEM((2, S_half, N), jnp.bfloat16),
            pltpu.VMEM((2, S_half, N), jnp.bfloat16),
            pltpu.VMEM((S_half, N), jnp.bfloat16),
            pltpu.VMEM((S_half, N), jnp.bfloat16),
            pltpu.SemaphoreType.DMA((3,)),
            pltpu.SemaphoreType.DMA((3,)),
            pltpu.SemaphoreType.DMA((4,)),
            pltpu.SemaphoreType.DMA((4,)),
        ],
        compiler_params=pltpu.CompilerParams(collective_id=0),
    )(O2, Wo)
